# baseline (speedup 1.0000x reference)
import functools

import jax
import jax.numpy as jnp
from jax import lax
from jax.experimental import pallas as pl
from jax.experimental.pallas import tpu as pltpu
from jax.experimental.pallas import tpu_sc as plsc

VOCAB = 100000
EMBED = 64
MAX_WORD_LEN = 50
HIDDEN = 1024
N_CLASSES = 128
BATCH = 4096

BL = BATCH * MAX_WORD_LEN
PAIRS = MAX_WORD_LEN // 2
YROWS = BL // 2


def _sc_worker_count():
    try:
        info = plsc.get_sparse_core_info()
        return info.num_cores * info.num_subcores
    except Exception:
        return 32


@functools.lru_cache(maxsize=None)
def _make_sc_gather(n_workers: int, chunk: int):
    per_w = YROWS // n_workers
    n_chunks = per_w // chunk
    mesh = plsc.VectorSubcoreMesh(core_axis_name="c", subcore_axis_name="s")

    @functools.partial(
        pl.kernel,
        out_type=jax.ShapeDtypeStruct((YROWS, 2 * EMBED), jnp.bfloat16),
        mesh=mesh,
        scratch_types=[
            pltpu.VMEM((chunk,), jnp.int32),
            pltpu.VMEM((chunk, EMBED), jnp.bfloat16),
            pltpu.SemaphoreType.DMA,
        ],
        compiler_params=pltpu.CompilerParams(use_tc_tiling_on_sc=False),
    )
    def gather_kernel(idx_hbm, table_hbm, out_hbm, idx_v, rows_v, sem):
        wid = lax.axis_index("s") * 2 + lax.axis_index("c")
        base = wid * per_w
        for c in range(n_chunks):
            off = base + c * chunk
            for h in range(2):
                pltpu.sync_copy(idx_hbm.at[h, pl.ds(off, chunk)], idx_v)
                pltpu.async_copy(table_hbm.at[idx_v], rows_v, sem).wait()
                pltpu.sync_copy(
                    rows_v,
                    out_hbm.at[pl.ds(off, chunk), pl.ds(h * EMBED, EMBED)])

    return gather_kernel


def _mlp_body(z_ref, w1_ref, b1_ref, w2_ref, b2_ref, out_ref):
    z = z_ref[...]
    a = jnp.concatenate([z[t] for t in range(PAIRS)], axis=1)
    h = lax.dot_general(a, w1_ref[...], (((1,), (1,)), ((), ())),
                        preferred_element_type=jnp.float32)
    h = jnp.maximum(h + b1_ref[...], 0.0).astype(jnp.bfloat16)
    out_ref[...] = lax.dot_general(h, w2_ref[...], (((1,), (1,)), ((), ())),
                                   preferred_element_type=jnp.float32) + b2_ref[...]


@functools.lru_cache(maxsize=None)
def _make_mlp(tile_b: int):
    in_dim = MAX_WORD_LEN * EMBED
    grid = (BATCH // tile_b,)
    return pl.pallas_call(
        _mlp_body,
        grid=grid,
        in_specs=[
            pl.BlockSpec((PAIRS, tile_b, 2 * EMBED), lambda i: (0, i, 0)),
            pl.BlockSpec((HIDDEN, in_dim), lambda i: (0, 0)),
            pl.BlockSpec((1, HIDDEN), lambda i: (0, 0)),
            pl.BlockSpec((N_CLASSES, HIDDEN), lambda i: (0, 0)),
            pl.BlockSpec((1, N_CLASSES), lambda i: (0, 0)),
        ],
        out_specs=pl.BlockSpec((tile_b, N_CLASSES), lambda i: (i, 0)),
        out_shape=jax.ShapeDtypeStruct((BATCH, N_CLASSES), jnp.float32),
    )


def kernel(x, table, W1, b1, W2, b2):
    nw = _sc_worker_count()
    idx2 = x.astype(jnp.int32).reshape(BATCH, PAIRS, 2).transpose(2, 1, 0).reshape(2, YROWS)
    y2 = _make_sc_gather(nw, 800)(idx2, table.astype(jnp.bfloat16))
    z = y2.reshape(PAIRS, BATCH, 2 * EMBED)
    out = _make_mlp(512)(
        z, W1.astype(jnp.bfloat16), b1.reshape(1, HIDDEN),
        W2.astype(jnp.bfloat16), b2.reshape(1, N_CLASSES))
    return out


# speedup vs baseline: 1.7699x; 1.7699x over previous
import functools

import jax
import jax.numpy as jnp
from jax import lax
from jax.experimental import pallas as pl
from jax.experimental.pallas import tpu as pltpu
from jax.experimental.pallas import tpu_sc as plsc

VOCAB = 100000
EMBED = 64
MAX_WORD_LEN = 50
HIDDEN = 1024
N_CLASSES = 128
BATCH = 4096

BL = BATCH * MAX_WORD_LEN
PAIRS = MAX_WORD_LEN // 2
YROWS = BL // 2
IDX_COLS = 12800


def _sc_worker_count():
    try:
        info = plsc.get_sparse_core_info()
        return info.num_cores * info.num_subcores
    except Exception:
        return 32


@functools.lru_cache(maxsize=None)
def _make_sc_gather(n_workers: int, chunk: int):
    per_w = YROWS // n_workers
    n_chunks = per_w // chunk
    mesh = plsc.VectorSubcoreMesh(core_axis_name="c", subcore_axis_name="s")

    @functools.partial(
        pl.kernel,
        out_type=jax.ShapeDtypeStruct((YROWS, 2 * EMBED), jnp.float32),
        mesh=mesh,
        scratch_types=[
            pltpu.VMEM((chunk,), jnp.int32),
            pltpu.VMEM((chunk, EMBED), jnp.float32),
            pltpu.SemaphoreType.DMA,
        ],
        compiler_params=pltpu.CompilerParams(use_tc_tiling_on_sc=False),
    )
    def gather_kernel(idx_hbm, table_hbm, out_hbm, idx_v, rows_v, sem):
        wid = lax.axis_index("s") * 2 + lax.axis_index("c")
        base = wid * per_w
        for c in range(n_chunks):
            off = base + c * chunk
            for h in range(2):
                flat = h * YROWS + off
                pltpu.sync_copy(
                    idx_hbm.at[flat // IDX_COLS, pl.ds(flat % IDX_COLS, chunk)],
                    idx_v)
                pltpu.async_copy(table_hbm.at[idx_v], rows_v, sem).wait()
                pltpu.sync_copy(
                    rows_v,
                    out_hbm.at[pl.ds(off, chunk), pl.ds(h * EMBED, EMBED)])

    return gather_kernel


def _mlp_body(z_ref, w1_ref, b1_ref, w2_ref, b2_ref, out_ref):
    z = z_ref[...]
    a = jnp.concatenate([z[t] for t in range(PAIRS)], axis=1)
    h = lax.dot_general(a, w1_ref[...], (((1,), (1,)), ((), ())),
                        preferred_element_type=jnp.float32)
    h = jnp.maximum(h + b1_ref[...], 0.0)
    out_ref[...] = lax.dot_general(h, w2_ref[...], (((1,), (1,)), ((), ())),
                                   preferred_element_type=jnp.float32) + b2_ref[...]


@functools.lru_cache(maxsize=None)
def _make_mlp(tile_b: int):
    in_dim = MAX_WORD_LEN * EMBED
    grid = (BATCH // tile_b,)
    return pl.pallas_call(
        _mlp_body,
        grid=grid,
        in_specs=[
            pl.BlockSpec((PAIRS, tile_b, 2 * EMBED), lambda i: (0, i, 0)),
            pl.BlockSpec((HIDDEN, in_dim), lambda i: (0, 0)),
            pl.BlockSpec((1, HIDDEN), lambda i: (0, 0)),
            pl.BlockSpec((N_CLASSES, HIDDEN), lambda i: (0, 0)),
            pl.BlockSpec((1, N_CLASSES), lambda i: (0, 0)),
        ],
        out_specs=pl.BlockSpec((tile_b, N_CLASSES), lambda i: (i, 0)),
        out_shape=jax.ShapeDtypeStruct((BATCH, N_CLASSES), jnp.float32),
    )


def kernel(x, table, W1, b1, W2, b2):
    nw = _sc_worker_count()
    idx2 = x.astype(jnp.int32).reshape(BATCH, PAIRS, 2).transpose(2, 1, 0)
    idx16 = idx2.reshape(16, IDX_COLS)
    y2 = _make_sc_gather(nw, 800)(idx16, table)
    z = y2.reshape(PAIRS, BATCH, 2 * EMBED)
    out = _make_mlp(512)(
        z, W1, b1.reshape(1, HIDDEN), W2, b2.reshape(1, N_CLASSES))
    return out


# speedup vs baseline: 1.7991x; 1.0165x over previous
import functools

import jax
import jax.numpy as jnp
from jax import lax
from jax.experimental import pallas as pl
from jax.experimental.pallas import tpu as pltpu
from jax.experimental.pallas import tpu_sc as plsc

VOCAB = 100000
EMBED = 64
MAX_WORD_LEN = 50
HIDDEN = 1024
N_CLASSES = 128
BATCH = 4096

NCHUNK = 2
BCH = BATCH // NCHUNK
PAIRS = MAX_WORD_LEN // 2
YROWS = BCH * MAX_WORD_LEN // 2
IDX_COLS = 12800


def _sc_worker_count():
    try:
        info = plsc.get_sparse_core_info()
        return info.num_cores * info.num_subcores
    except Exception:
        return 32


@functools.lru_cache(maxsize=None)
def _make_sc_gather(n_workers: int, chunk: int):
    per_w = YROWS // n_workers
    n_chunks = per_w // chunk
    mesh = plsc.VectorSubcoreMesh(core_axis_name="c", subcore_axis_name="s")

    @functools.partial(
        pl.kernel,
        out_type=jax.ShapeDtypeStruct((YROWS, 2 * EMBED), jnp.float32),
        mesh=mesh,
        scratch_types=[
            pltpu.VMEM((chunk,), jnp.int32),
            pltpu.VMEM((chunk, EMBED), jnp.float32),
            pltpu.SemaphoreType.DMA,
        ],
        compiler_params=pltpu.CompilerParams(use_tc_tiling_on_sc=False),
    )
    def gather_kernel(idx_hbm, table_hbm, out_hbm, idx_v, rows_v, sem):
        wid = lax.axis_index("s") * 2 + lax.axis_index("c")
        base = wid * per_w
        for c in range(n_chunks):
            off = base + c * chunk
            for h in range(2):
                flat = h * YROWS + off
                pltpu.sync_copy(
                    idx_hbm.at[flat // IDX_COLS, pl.ds(flat % IDX_COLS, chunk)],
                    idx_v)
                pltpu.async_copy(table_hbm.at[idx_v], rows_v, sem).wait()
                pltpu.sync_copy(
                    rows_v,
                    out_hbm.at[pl.ds(off, chunk), pl.ds(h * EMBED, EMBED)])

    return gather_kernel


def _mlp_body(z_ref, w1_ref, b1_ref, w2_ref, b2_ref, out_ref):
    z = z_ref[...]
    a = jnp.concatenate([z[t] for t in range(PAIRS)], axis=1)
    h = lax.dot_general(a, w1_ref[...], (((1,), (1,)), ((), ())),
                        preferred_element_type=jnp.float32)
    h = jnp.maximum(h + b1_ref[...], 0.0)
    out_ref[...] = lax.dot_general(h, w2_ref[...], (((1,), (1,)), ((), ())),
                                   preferred_element_type=jnp.float32) + b2_ref[...]


@functools.lru_cache(maxsize=None)
def _make_mlp(tile_b: int):
    in_dim = MAX_WORD_LEN * EMBED
    grid = (BCH // tile_b,)
    return pl.pallas_call(
        _mlp_body,
        grid=grid,
        in_specs=[
            pl.BlockSpec((PAIRS, tile_b, 2 * EMBED), lambda i: (0, i, 0)),
            pl.BlockSpec((HIDDEN, in_dim), lambda i: (0, 0)),
            pl.BlockSpec((1, HIDDEN), lambda i: (0, 0)),
            pl.BlockSpec((N_CLASSES, HIDDEN), lambda i: (0, 0)),
            pl.BlockSpec((1, N_CLASSES), lambda i: (0, 0)),
        ],
        out_specs=pl.BlockSpec((tile_b, N_CLASSES), lambda i: (i, 0)),
        out_shape=jax.ShapeDtypeStruct((BCH, N_CLASSES), jnp.float32),
    )


def kernel(x, table, W1, b1, W2, b2):
    nw = _sc_worker_count()
    xi = x.astype(jnp.int32)
    b1r = b1.reshape(1, HIDDEN)
    b2r = b2.reshape(1, N_CLASSES)
    outs = []
    for c in range(NCHUNK):
        xc = xi[c * BCH:(c + 1) * BCH]
        idx2 = xc.reshape(BCH, PAIRS, 2).transpose(2, 1, 0)
        idx16 = idx2.reshape(2 * YROWS // IDX_COLS, IDX_COLS)
        y2 = _make_sc_gather(nw, 800)(idx16, table)
        z = y2.reshape(PAIRS, BCH, 2 * EMBED)
        outs.append(_make_mlp(512)(z, W1, b1r, W2, b2r))
    return jnp.concatenate(outs, axis=0)
